# baseline (speedup 1.0000x reference)
import jax
import jax.numpy as jnp
from jax import lax
from jax.experimental import pallas as pl
from jax.experimental.pallas import tpu as pltpu
from jax.experimental.pallas import tpu_sc as plsc

B = 2048
D = 64
K = 1024
NC = 2
NS = 16
NW = NC * NS
BPW = B // NW
NG = BPW // 16
DPS = D // NS


def _som_sc_body(x_hbm, w_hbm, loc_hbm, loc_out, mind2_out, stats_hbm,
                 x_v, w_v, loc_v, stats_loc, stats_all, m_arr, v_arr,
                 outloc_v, mind2_v, sem, sem2):
    c = lax.axis_index("c")
    s = lax.axis_index("s")
    wid = c * NS + s
    base = wid * BPW

    cp_x = pltpu.async_copy(x_hbm.at[pl.ds(base, BPW)], x_v, sem)
    cp_loc = pltpu.async_copy(loc_hbm.at[pl.ds(0, D)], loc_v, sem2)

    pltpu.sync_copy(w_hbm.at[pl.ds(s * DPS, DPS)], w_v)

    zero16 = jnp.zeros((16,), jnp.float32)

    @plsc.parallel_loop(0, K // 16, unroll=4, carry=(zero16,) * (2 * DPS))
    def _stats_loop(i, carry):
        accs = list(carry)
        off = i * 16
        for j in range(DPS):
            v = w_v[j, pl.ds(off, 16)]
            accs[2 * j] = accs[2 * j] + v
            accs[2 * j + 1] = accs[2 * j + 1] + v * v

        return tuple(accs)

    accs = _stats_loop
    lane = lax.iota(jnp.int32, 16)
    stats_vec = zero16
    for j in range(DPS):
        s_sum = jnp.sum(accs[2 * j])
        q_sum = jnp.sum(accs[2 * j + 1])
        m_d = s_sum * (1.0 / K)
        v_d = q_sum - s_sum * m_d
        stats_vec = jnp.where(lane == 2 * j, m_d, stats_vec)
        stats_vec = jnp.where(lane == 2 * j + 1, v_d, stats_vec)
    stats_loc[...] = stats_vec
    pltpu.sync_copy(stats_loc, stats_hbm.at[s])
    plsc.subcore_barrier()
    pltpu.sync_copy(stats_hbm, stats_all)

    for ch in range(D // 16):
        dvec = lane + ch * 16
        rowi = lax.shift_right_logical(dvec, 2)
        coli = lax.shift_left(jnp.bitwise_and(dvec, 3), 1)
        m_arr[pl.ds(ch * 16, 16)] = plsc.load_gather(stats_all, [rowi, coli])
        v_arr[pl.ds(ch * 16, 16)] = plsc.load_gather(stats_all, [rowi, coli + 1])

    cp_x.wait()
    cp_loc.wait()

    rows = [lane + 16 * g for g in range(NG)]
    inf16 = jnp.full((16,), jnp.inf, jnp.float32)
    izero16 = jnp.zeros((16,), jnp.int32)

    @plsc.parallel_loop(0, D, unroll=8,
                        carry=(inf16,) * NG + (izero16,) * NG)
    def _search_loop(d, carry):
        vmins = list(carry[:NG])
        vidxs = list(carry[NG:])
        dfull = izero16 + d
        mvec = plsc.load_gather(m_arr, [dfull])
        vvec = plsc.load_gather(v_arr, [dfull])
        for g in range(NG):
            xc = plsc.load_gather(x_v, [rows[g], dfull])
            t = (xc - mvec) + 1e-6
            dist2 = (t * t) * float(K) + vvec
            lt = dist2 < vmins[g]
            vmins[g] = jnp.where(lt, dist2, vmins[g])
            vidxs[g] = jnp.where(lt, dfull, vidxs[g])
        return tuple(vmins) + tuple(vidxs)

    carry = _search_loop
    vmins, vidxs = carry[:NG], carry[NG:]

    ones = izero16 + 1
    for g in range(NG):
        gx = plsc.load_gather(loc_v, [vidxs[g], izero16])
        gy = plsc.load_gather(loc_v, [vidxs[g], ones])
        plsc.store_scatter(outloc_v, [rows[g], izero16, izero16], gx)
        plsc.store_scatter(outloc_v, [rows[g], izero16, ones], gy)
        mind2_v[pl.ds(16 * g, 16)] = vmins[g]
    cp_o1 = pltpu.async_copy(outloc_v, loc_out.at[pl.ds(base, BPW)], sem)
    cp_o2 = pltpu.async_copy(mind2_v, mind2_out.at[pl.ds(base, BPW)], sem2)
    cp_o1.wait()
    cp_o2.wait()


_som_sc = pl.kernel(
    _som_sc_body,
    out_type=[
        jax.ShapeDtypeStruct((B, 1, 2), jnp.float32),
        jax.ShapeDtypeStruct((B,), jnp.float32),
        jax.ShapeDtypeStruct((NS, 16), jnp.float32),
    ],
    mesh=plsc.VectorSubcoreMesh(core_axis_name="c", subcore_axis_name="s",
                                num_cores=NC, num_subcores=NS),
    compiler_params=pltpu.CompilerParams(needs_layout_passes=False),
    scratch_types=[
        pltpu.VMEM((BPW, D), jnp.float32),
        pltpu.VMEM((DPS, K), jnp.float32),
        pltpu.VMEM((D, 2), jnp.float32),
        pltpu.VMEM((16,), jnp.float32),
        pltpu.VMEM((NS, 16), jnp.float32),
        pltpu.VMEM((D,), jnp.float32),
        pltpu.VMEM((D,), jnp.float32),
        pltpu.VMEM((BPW, 1, 2), jnp.float32),
        pltpu.VMEM((BPW,), jnp.float32),
        pltpu.SemaphoreType.DMA,
        pltpu.SemaphoreType.DMA,
    ],
)


def _loss_body(d2_ref, o_ref):
    total = jnp.sum(jnp.sqrt(d2_ref[...])) * (1.0 / B)
    o_ref[...] = total.reshape(1, 1)


_loss_tc = pl.pallas_call(
    _loss_body,
    out_shape=jax.ShapeDtypeStruct((1, 1), jnp.float32),
)


def kernel(input, weight, locations):
    bmu, mind2, _ = _som_sc(input, weight, locations)
    loss = _loss_tc(mind2.reshape(16, 128))
    return bmu, loss[0, 0]


# speedup vs baseline: 1.0024x; 1.0024x over previous
import jax
import jax.numpy as jnp
from jax import lax
from jax.experimental import pallas as pl
from jax.experimental.pallas import tpu as pltpu
from jax.experimental.pallas import tpu_sc as plsc

B = 2048
D = 64
K = 1024
NC = 2
NS = 16
NW = NC * NS
BPW = B // NW
NG = BPW // 16
DPS = D // NS


def _som_sc_body(x_hbm, w_hbm, loc_hbm, loc_out, mind2_out, stats_hbm,
                 x_v, w_v, loc_v, stats_loc, stats_all, m_arr, v_arr,
                 outloc_v, mind2_v, sem, sem2):
    c = lax.axis_index("c")
    s = lax.axis_index("s")
    wid = c * NS + s
    base = wid * BPW

    cp_x = pltpu.async_copy(x_hbm.at[pl.ds(base, BPW)], x_v, sem)
    cp_loc = pltpu.async_copy(loc_hbm.at[pl.ds(0, D)], loc_v, sem2)

    pltpu.sync_copy(w_hbm.at[pl.ds(s * DPS, DPS)], w_v)

    zero16 = jnp.zeros((16,), jnp.float32)

    @plsc.parallel_loop(0, K // 16, unroll=4, carry=(zero16,) * (2 * DPS))
    def _stats_loop(i, carry):
        accs = list(carry)
        off = i * 16
        for j in range(DPS):
            v = w_v[j, pl.ds(off, 16)]
            accs[2 * j] = accs[2 * j] + v
            accs[2 * j + 1] = accs[2 * j + 1] + v * v

        return tuple(accs)

    accs = _stats_loop
    lane = lax.iota(jnp.int32, 16)
    stats_vec = zero16
    for j in range(DPS):
        s_sum = jnp.sum(accs[2 * j])
        q_sum = jnp.sum(accs[2 * j + 1])
        m_d = s_sum * (1.0 / K)
        v_d = q_sum - s_sum * m_d
        stats_vec = jnp.where(lane == 2 * j, m_d, stats_vec)
        stats_vec = jnp.where(lane == 2 * j + 1, v_d, stats_vec)
    stats_loc[...] = stats_vec
    pltpu.sync_copy(stats_loc, stats_hbm.at[s])
    plsc.subcore_barrier()
    pltpu.sync_copy(stats_hbm, stats_all)

    for ch in range(D // 16):
        dvec = lane + ch * 16
        rowi = lax.shift_right_logical(dvec, 2)
        coli = lax.shift_left(jnp.bitwise_and(dvec, 3), 1)
        m_arr[pl.ds(ch * 16, 16)] = plsc.load_gather(stats_all, [rowi, coli])
        v_arr[pl.ds(ch * 16, 16)] = plsc.load_gather(stats_all, [rowi, coli + 1])

    cp_x.wait()
    cp_loc.wait()

    rows = [lane + 16 * g for g in range(NG)]
    inf16 = jnp.full((16,), jnp.inf, jnp.float32)
    izero16 = jnp.zeros((16,), jnp.int32)

    @plsc.parallel_loop(0, D, unroll=4,
                        carry=(inf16,) * NG + (izero16,) * NG)
    def _search_loop(d, carry):
        vmins = list(carry[:NG])
        vidxs = list(carry[NG:])
        dfull = izero16 + d
        mvec = plsc.load_gather(m_arr, [dfull])
        vvec = plsc.load_gather(v_arr, [dfull])
        for g in range(NG):
            xc = plsc.load_gather(x_v, [rows[g], dfull])
            t = (xc - mvec) + 1e-6
            dist2 = (t * t) * float(K) + vvec
            lt = dist2 < vmins[g]
            vmins[g] = jnp.where(lt, dist2, vmins[g])
            vidxs[g] = jnp.where(lt, dfull, vidxs[g])
        return tuple(vmins) + tuple(vidxs)

    carry = _search_loop
    vmins, vidxs = carry[:NG], carry[NG:]

    ones = izero16 + 1
    for g in range(NG):
        gx = plsc.load_gather(loc_v, [vidxs[g], izero16])
        gy = plsc.load_gather(loc_v, [vidxs[g], ones])
        plsc.store_scatter(outloc_v, [rows[g], izero16, izero16], gx)
        plsc.store_scatter(outloc_v, [rows[g], izero16, ones], gy)
        mind2_v[pl.ds(16 * g, 16)] = vmins[g]
    cp_o1 = pltpu.async_copy(outloc_v, loc_out.at[pl.ds(base, BPW)], sem)
    cp_o2 = pltpu.async_copy(mind2_v, mind2_out.at[pl.ds(base, BPW)], sem2)
    cp_o1.wait()
    cp_o2.wait()


_som_sc = pl.kernel(
    _som_sc_body,
    out_type=[
        jax.ShapeDtypeStruct((B, 1, 2), jnp.float32),
        jax.ShapeDtypeStruct((B,), jnp.float32),
        jax.ShapeDtypeStruct((NS, 16), jnp.float32),
    ],
    mesh=plsc.VectorSubcoreMesh(core_axis_name="c", subcore_axis_name="s",
                                num_cores=NC, num_subcores=NS),
    compiler_params=pltpu.CompilerParams(needs_layout_passes=False),
    scratch_types=[
        pltpu.VMEM((BPW, D), jnp.float32),
        pltpu.VMEM((DPS, K), jnp.float32),
        pltpu.VMEM((D, 2), jnp.float32),
        pltpu.VMEM((16,), jnp.float32),
        pltpu.VMEM((NS, 16), jnp.float32),
        pltpu.VMEM((D,), jnp.float32),
        pltpu.VMEM((D,), jnp.float32),
        pltpu.VMEM((BPW, 1, 2), jnp.float32),
        pltpu.VMEM((BPW,), jnp.float32),
        pltpu.SemaphoreType.DMA,
        pltpu.SemaphoreType.DMA,
    ],
)


def _loss_body(d2_ref, o_ref):
    total = jnp.sum(jnp.sqrt(d2_ref[...])) * (1.0 / B)
    o_ref[...] = total.reshape(1, 1)


_loss_tc = pl.pallas_call(
    _loss_body,
    out_shape=jax.ShapeDtypeStruct((1, 1), jnp.float32),
)


def kernel(input, weight, locations):
    bmu, mind2, _ = _som_sc(input, weight, locations)
    loss = _loss_tc(mind2.reshape(16, 128))
    return bmu, loss[0, 0]


# speedup vs baseline: 1.0379x; 1.0355x over previous
import jax
import jax.numpy as jnp
from jax import lax
from jax.experimental import pallas as pl
from jax.experimental.pallas import tpu as pltpu
from jax.experimental.pallas import tpu_sc as plsc

B = 2048
D = 64
K = 1024
NC = 2
NS = 16
NW = NC * NS
BPW = B // NW
NG = BPW // 16
DPS = D // NS


def _som_sc_body(x_hbm, w_hbm, loc_hbm, loc_out, mind2_out, stats_hbm,
                 x_v, w_v, loc_v, stats_loc, stats_all, m_arr, v_arr,
                 outloc_v, mind2_v, sem, sem2):
    c = lax.axis_index("c")
    s = lax.axis_index("s")
    wid = c * NS + s
    base = wid * BPW

    cp_x = pltpu.async_copy(x_hbm.at[pl.ds(base, BPW)], x_v, sem)
    cp_loc = pltpu.async_copy(loc_hbm.at[pl.ds(0, D)], loc_v, sem2)

    pltpu.sync_copy(w_hbm.at[pl.ds(s * DPS, DPS)], w_v)

    zero16 = jnp.zeros((16,), jnp.float32)

    @plsc.parallel_loop(0, K // 16, unroll=4, carry=(zero16,) * (2 * DPS))
    def _stats_loop(i, carry):
        accs = list(carry)
        off = i * 16
        for j in range(DPS):
            v = w_v[j, pl.ds(off, 16)]
            accs[2 * j] = accs[2 * j] + v
            accs[2 * j + 1] = accs[2 * j + 1] + v * v

        return tuple(accs)

    accs = _stats_loop
    lane = lax.iota(jnp.int32, 16)
    stats_vec = zero16
    for j in range(DPS):
        s_sum = jnp.sum(accs[2 * j])
        q_sum = jnp.sum(accs[2 * j + 1])
        m_d = s_sum * (1.0 / K)
        v_d = q_sum - s_sum * m_d
        stats_vec = jnp.where(lane == 2 * j, m_d, stats_vec)
        stats_vec = jnp.where(lane == 2 * j + 1, v_d, stats_vec)
    stats_loc[...] = stats_vec
    pltpu.sync_copy(stats_loc, stats_hbm.at[s])
    plsc.subcore_barrier()
    pltpu.sync_copy(stats_hbm, stats_all)

    for ch in range(D // 16):
        dvec = lane + ch * 16
        rowi = lax.shift_right_logical(dvec, 2)
        coli = lax.shift_left(jnp.bitwise_and(dvec, 3), 1)
        m_arr[pl.ds(ch * 16, 16)] = plsc.load_gather(stats_all, [rowi, coli])
        v_arr[pl.ds(ch * 16, 16)] = plsc.load_gather(stats_all, [rowi, coli + 1])

    cp_x.wait()
    cp_loc.wait()

    rows = [lane + 16 * g for g in range(NG)]
    inf16 = jnp.full((16,), jnp.inf, jnp.float32)
    izero16 = jnp.zeros((16,), jnp.int32)

    @plsc.parallel_loop(0, D, unroll=4,
                        carry=(inf16,) * NG + (izero16,) * NG)
    def _search_loop(d, carry):
        vmins = list(carry[:NG])
        vidxs = list(carry[NG:])
        dfull = izero16 + d
        mvec = plsc.load_gather(m_arr, [dfull])
        vvec = plsc.load_gather(v_arr, [dfull])
        for g in range(NG):
            xc = plsc.load_gather(x_v, [rows[g], dfull])
            t = (xc - mvec) + 1e-6
            dist2 = (t * t) * float(K) + vvec
            lt = dist2 < vmins[g]
            vmins[g] = jnp.where(lt, dist2, vmins[g])
            vidxs[g] = jnp.where(lt, dfull, vidxs[g])
        return tuple(vmins) + tuple(vidxs)

    carry = _search_loop
    vmins, vidxs = carry[:NG], carry[NG:]

    ones = izero16 + 1
    for g in range(NG):
        gx = plsc.load_gather(loc_v, [vidxs[g], izero16])
        gy = plsc.load_gather(loc_v, [vidxs[g], ones])
        plsc.store_scatter(outloc_v, [rows[g], izero16], gx)
        plsc.store_scatter(outloc_v, [rows[g], ones], gy)
        mind2_v[pl.ds(16 * g, 16)] = vmins[g]
    cp_o1 = pltpu.async_copy(outloc_v, loc_out.at[pl.ds(base, BPW)], sem)
    cp_o2 = pltpu.async_copy(mind2_v, mind2_out.at[pl.ds(base, BPW)], sem2)
    cp_o1.wait()
    cp_o2.wait()


_som_sc = pl.kernel(
    _som_sc_body,
    out_type=[
        jax.ShapeDtypeStruct((B, 2), jnp.float32),
        jax.ShapeDtypeStruct((B,), jnp.float32),
        jax.ShapeDtypeStruct((NS, 16), jnp.float32),
    ],
    mesh=plsc.VectorSubcoreMesh(core_axis_name="c", subcore_axis_name="s",
                                num_cores=NC, num_subcores=NS),
    compiler_params=pltpu.CompilerParams(needs_layout_passes=False),
    scratch_types=[
        pltpu.VMEM((BPW, D), jnp.float32),
        pltpu.VMEM((DPS, K), jnp.float32),
        pltpu.VMEM((D, 2), jnp.float32),
        pltpu.VMEM((16,), jnp.float32),
        pltpu.VMEM((NS, 16), jnp.float32),
        pltpu.VMEM((D,), jnp.float32),
        pltpu.VMEM((D,), jnp.float32),
        pltpu.VMEM((BPW, 2), jnp.float32),
        pltpu.VMEM((BPW,), jnp.float32),
        pltpu.SemaphoreType.DMA,
        pltpu.SemaphoreType.DMA,
    ],
)


def _loss_body(d2_ref, o_ref):
    total = jnp.sum(jnp.sqrt(d2_ref[...])) * (1.0 / B)
    o_ref[...] = total.reshape(1, 1)


_loss_tc = pl.pallas_call(
    _loss_body,
    out_shape=jax.ShapeDtypeStruct((1, 1), jnp.float32),
)


def kernel(input, weight, locations):
    bmu2, mind2, _ = _som_sc(input, weight, locations)
    loss = _loss_tc(mind2.reshape(16, 128))
    return bmu2.reshape(B, 1, 2), loss[0, 0]


# speedup vs baseline: 1.0407x; 1.0027x over previous
import jax
import jax.numpy as jnp
from jax import lax
from jax.experimental import pallas as pl
from jax.experimental.pallas import tpu as pltpu
from jax.experimental.pallas import tpu_sc as plsc

B = 2048
D = 64
K = 1024
NC = 2
NS = 16
NW = NC * NS
BPW = B // NW
NG = BPW // 16
DPS = D // NS


def _som_sc_body(x_hbm, w_hbm, loc_hbm, loc_out, mind2_out, stats_hbm,
                 x_v, w_v, loc_v, stats_loc, stats_all, m_arr, v_arr,
                 outloc_v, mind2_v, sem, sem2):
    c = lax.axis_index("c")
    s = lax.axis_index("s")
    wid = c * NS + s
    base = wid * BPW

    cp_x = pltpu.async_copy(x_hbm.at[pl.ds(base, BPW)], x_v, sem)
    cp_loc = pltpu.async_copy(loc_hbm.at[pl.ds(0, D)], loc_v, sem2)

    pltpu.sync_copy(w_hbm.at[pl.ds(s * DPS, DPS)], w_v)

    zero16 = jnp.zeros((16,), jnp.float32)

    @plsc.parallel_loop(0, K // 16, unroll=4, carry=(zero16,) * (2 * DPS))
    def _stats_loop(i, carry):
        accs = list(carry)
        off = i * 16
        for j in range(DPS):
            v = w_v[j, pl.ds(off, 16)]
            accs[2 * j] = accs[2 * j] + v
            accs[2 * j + 1] = accs[2 * j + 1] + v * v

        return tuple(accs)

    accs = _stats_loop
    lane = lax.iota(jnp.int32, 16)
    stats_vec = zero16
    for j in range(DPS):
        s_sum = jnp.sum(accs[2 * j])
        q_sum = jnp.sum(accs[2 * j + 1])
        m_d = s_sum * (1.0 / K)
        v_d = q_sum - s_sum * m_d
        stats_vec = jnp.where(lane == 2 * j, m_d, stats_vec)
        stats_vec = jnp.where(lane == 2 * j + 1, v_d, stats_vec)
    stats_loc[...] = stats_vec
    pltpu.sync_copy(stats_loc, stats_hbm.at[s])
    plsc.subcore_barrier()
    pltpu.sync_copy(stats_hbm, stats_all)

    for ch in range(D // 16):
        dvec = lane + ch * 16
        rowi = lax.shift_right_logical(dvec, 2)
        coli = lax.shift_left(jnp.bitwise_and(dvec, 3), 1)
        m_arr[pl.ds(ch * 16, 16)] = plsc.load_gather(stats_all, [rowi, coli])
        v_arr[pl.ds(ch * 16, 16)] = plsc.load_gather(stats_all, [rowi, coli + 1])

    cp_x.wait()
    cp_loc.wait()

    rows = [lane + 16 * g for g in range(NG)]
    inf16 = jnp.full((16,), jnp.inf, jnp.float32)
    izero16 = jnp.zeros((16,), jnp.int32)

    @plsc.parallel_loop(0, D, unroll=4,
                        carry=(inf16,) * NG + (izero16,) * NG)
    def _search_loop(d, carry):
        vmins = list(carry[:NG])
        vidxs = list(carry[NG:])
        dfull = izero16 + d
        mvec = plsc.load_gather(m_arr, [dfull])
        vvec = plsc.load_gather(v_arr, [dfull])
        for g in range(NG):
            xc = plsc.load_gather(x_v, [rows[g], dfull])
            t = (xc - mvec) + 1e-6
            dist2 = (t * t) * float(K) + vvec
            lt = dist2 < vmins[g]
            vmins[g] = jnp.where(lt, dist2, vmins[g])
            vidxs[g] = jnp.where(lt, dfull, vidxs[g])
        return tuple(vmins) + tuple(vidxs)

    carry = _search_loop
    vmins, vidxs = carry[:NG], carry[NG:]

    ones = izero16 + 1
    for g in range(NG):
        gx = plsc.load_gather(loc_v, [vidxs[g], izero16])
        gy = plsc.load_gather(loc_v, [vidxs[g], ones])
        plsc.store_scatter(outloc_v, [rows[g], izero16], gx)
        plsc.store_scatter(outloc_v, [rows[g], ones], gy)
        mind2_v[pl.ds(16 * g, 16)] = vmins[g]
    cp_o1 = pltpu.async_copy(outloc_v, loc_out.at[pl.ds(base, BPW)], sem)
    cp_o2 = pltpu.async_copy(mind2_v, mind2_out.at[pl.ds(base, BPW)], sem2)
    cp_o1.wait()
    cp_o2.wait()


_som_sc = pl.kernel(
    _som_sc_body,
    out_type=[
        jax.ShapeDtypeStruct((B, 2), jnp.float32),
        jax.ShapeDtypeStruct((B,), jnp.float32),
        jax.ShapeDtypeStruct((NS, 16), jnp.float32),
    ],
    mesh=plsc.VectorSubcoreMesh(core_axis_name="c", subcore_axis_name="s",
                                num_cores=NC, num_subcores=NS),
    compiler_params=pltpu.CompilerParams(needs_layout_passes=False,
                                         disable_bounds_checks=True,
                                         skip_device_barrier=True),
    scratch_types=[
        pltpu.VMEM((BPW, D), jnp.float32),
        pltpu.VMEM((DPS, K), jnp.float32),
        pltpu.VMEM((D, 2), jnp.float32),
        pltpu.VMEM((16,), jnp.float32),
        pltpu.VMEM((NS, 16), jnp.float32),
        pltpu.VMEM((D,), jnp.float32),
        pltpu.VMEM((D,), jnp.float32),
        pltpu.VMEM((BPW, 2), jnp.float32),
        pltpu.VMEM((BPW,), jnp.float32),
        pltpu.SemaphoreType.DMA,
        pltpu.SemaphoreType.DMA,
    ],
)


def _loss_body(d2_ref, o_ref):
    total = jnp.sum(jnp.sqrt(d2_ref[...])) * (1.0 / B)
    o_ref[...] = total.reshape(1, 1)


_loss_tc = pl.pallas_call(
    _loss_body,
    out_shape=jax.ShapeDtypeStruct((1, 1), jnp.float32),
)


def kernel(input, weight, locations):
    bmu2, mind2, _ = _som_sc(input, weight, locations)
    loss = _loss_tc(mind2.reshape(16, 128))
    return bmu2.reshape(B, 1, 2), loss[0, 0]


# speedup vs baseline: 1.0902x; 1.0476x over previous
import jax
import jax.numpy as jnp
from jax import lax
from jax.experimental import pallas as pl
from jax.experimental.pallas import tpu as pltpu
from jax.experimental.pallas import tpu_sc as plsc

B = 2048
D = 64
K = 1024
NC = 2
NS = 16
NW = NC * NS
BPW = B // NW
NG = BPW // 16
DPS = D // NS


def _som_sc_body(x_hbm, w_hbm, loc_hbm, loc_out, mind2_out, stats_hbm,
                 x_v, w_v, loc_v, stats_loc, stats_all, m_arr, v_arr,
                 x_sk, outloc_v, mind2_v, sem, sem2, sem3):
    c = lax.axis_index("c")
    s = lax.axis_index("s")
    wid = c * NS + s
    base = wid * BPW
    lane = lax.iota(jnp.int32, 16)
    izero16 = jnp.zeros((16,), jnp.int32)

    cp_x = pltpu.async_copy(x_hbm.at[pl.ds(base, BPW)], x_v, sem)
    cp_loc = pltpu.async_copy(loc_hbm.at[pl.ds(0, D)], loc_v, sem2)

    pltpu.sync_copy(w_hbm.at[pl.ds(s * DPS, DPS)], w_v)

    zero16 = jnp.zeros((16,), jnp.float32)

    @plsc.parallel_loop(0, K // 16, unroll=4, carry=(zero16,) * (2 * DPS))
    def _stats_loop(i, carry):
        accs = list(carry)
        off = i * 16
        for j in range(DPS):
            v = w_v[j, pl.ds(off, 16)]
            accs[2 * j] = accs[2 * j] + v
            accs[2 * j + 1] = accs[2 * j + 1] + v * v

        return tuple(accs)

    accs = _stats_loop
    stats_vec = zero16
    for j in range(DPS):
        s_sum = jnp.sum(accs[2 * j])
        q_sum = jnp.sum(accs[2 * j + 1])
        m_d = s_sum * (1.0 / K)
        v_d = q_sum - s_sum * m_d
        stats_vec = jnp.where(lane == 2 * j, m_d, stats_vec)
        stats_vec = jnp.where(lane == 2 * j + 1, v_d, stats_vec)
    stats_loc[...] = stats_vec
    cp_st = pltpu.async_copy(stats_loc, stats_hbm.at[s], sem3)

    cp_x.wait()

    @plsc.parallel_loop(0, BPW, unroll=2)
    def _skew_loop(b):
        bfull = izero16 + b
        for c2 in range(D // 16):
            colv = jnp.bitwise_and(lane + (16 * c2 - b), D - 1)
            gsk = plsc.load_gather(x_v, [bfull, colv])
            x_sk[pl.ds(b * D + 16 * c2, 16)] = gsk

    cp_st.wait()
    plsc.subcore_barrier()
    pltpu.sync_copy(stats_hbm, stats_all)

    for ch in range(D // 16):
        dvec = lane + ch * 16
        rowi = lax.shift_right_logical(dvec, 2)
        coli = lax.shift_left(jnp.bitwise_and(dvec, 3), 1)
        m_arr[pl.ds(ch * 16, 16)] = plsc.load_gather(stats_all, [rowi, coli])
        v_arr[pl.ds(ch * 16, 16)] = plsc.load_gather(stats_all, [rowi, coli + 1])

    cp_loc.wait()

    rows = [lane + 16 * g for g in range(NG)]
    rowbase = [rows[g] * D for g in range(NG)]
    inf16 = jnp.full((16,), jnp.inf, jnp.float32)

    @plsc.parallel_loop(0, D, unroll=4,
                        carry=(inf16,) * NG + (izero16,) * NG)
    def _search_loop(d, carry):
        vmins = list(carry[:NG])
        vidxs = list(carry[NG:])
        dfull = izero16 + d
        mvec = plsc.load_gather(m_arr, [dfull])
        vvec = plsc.load_gather(v_arr, [dfull])
        for g in range(NG):
            xc = plsc.load_gather(
                x_sk, [rowbase[g] + jnp.bitwise_and(dfull + rows[g], D - 1)])
            t = (xc - mvec) + 1e-6
            dist2 = (t * t) * float(K) + vvec
            lt = dist2 < vmins[g]
            vmins[g] = jnp.where(lt, dist2, vmins[g])
            vidxs[g] = jnp.where(lt, dfull, vidxs[g])
        return tuple(vmins) + tuple(vidxs)

    carry = _search_loop
    vmins, vidxs = carry[:NG], carry[NG:]

    ones = izero16 + 1
    for g in range(NG):
        gx = plsc.load_gather(loc_v, [vidxs[g], izero16])
        gy = plsc.load_gather(loc_v, [vidxs[g], ones])
        plsc.store_scatter(outloc_v, [rows[g], izero16], gx)
        plsc.store_scatter(outloc_v, [rows[g], ones], gy)
        mind2_v[pl.ds(16 * g, 16)] = vmins[g]
    cp_o1 = pltpu.async_copy(outloc_v, loc_out.at[pl.ds(base, BPW)], sem)
    cp_o2 = pltpu.async_copy(mind2_v, mind2_out.at[pl.ds(base, BPW)], sem2)
    cp_o1.wait()
    cp_o2.wait()


_som_sc = pl.kernel(
    _som_sc_body,
    out_type=[
        jax.ShapeDtypeStruct((B, 2), jnp.float32),
        jax.ShapeDtypeStruct((B,), jnp.float32),
        jax.ShapeDtypeStruct((NS, 16), jnp.float32),
    ],
    mesh=plsc.VectorSubcoreMesh(core_axis_name="c", subcore_axis_name="s",
                                num_cores=NC, num_subcores=NS),
    compiler_params=pltpu.CompilerParams(needs_layout_passes=False,
                                         disable_bounds_checks=True,
                                         skip_device_barrier=True),
    scratch_types=[
        pltpu.VMEM((BPW, D), jnp.float32),
        pltpu.VMEM((DPS, K), jnp.float32),
        pltpu.VMEM((D, 2), jnp.float32),
        pltpu.VMEM((16,), jnp.float32),
        pltpu.VMEM((NS, 16), jnp.float32),
        pltpu.VMEM((D,), jnp.float32),
        pltpu.VMEM((D,), jnp.float32),
        pltpu.VMEM((BPW * D,), jnp.float32),
        pltpu.VMEM((BPW, 2), jnp.float32),
        pltpu.VMEM((BPW,), jnp.float32),
        pltpu.SemaphoreType.DMA,
        pltpu.SemaphoreType.DMA,
        pltpu.SemaphoreType.DMA,
    ],
)


def _loss_body(d2_ref, o_ref):
    total = jnp.sum(jnp.sqrt(d2_ref[...])) * (1.0 / B)
    o_ref[...] = total.reshape(1, 1)


_loss_tc = pl.pallas_call(
    _loss_body,
    out_shape=jax.ShapeDtypeStruct((1, 1), jnp.float32),
)


def kernel(input, weight, locations):
    bmu2, mind2, _ = _som_sc(input, weight, locations)
    loss = _loss_tc(mind2.reshape(16, 128))
    return bmu2.reshape(B, 1, 2), loss[0, 0]
